# Initial kernel scaffold; baseline (speedup 1.0000x reference)
import jax
import jax.numpy as jnp
from jax.experimental import pallas as pl


def kernel(x, table):
    raise NotImplementedError("write your pallas kernel here")


# speedup vs baseline: 4.8115x; 4.8115x over previous
import functools

import jax
import jax.numpy as jnp
from jax import lax
from jax.experimental import pallas as pl
from jax.experimental.pallas import tpu as pltpu
from jax.experimental.pallas import tpu_sc as plsc

NUM_CORES = 2
NUM_SUBCORES = 16
NUM_WORKERS = NUM_CORES * NUM_SUBCORES
GROUP = 128


@functools.partial(jax.jit, static_argnames=("k", "d"))
def _sc_gather(idx, table, *, k, d):
    n_groups = idx.shape[0]
    groups_per_w = n_groups // NUM_WORKERS
    n_chunks = groups_per_w // k

    mesh = plsc.VectorSubcoreMesh(
        core_axis_name="c", subcore_axis_name="s",
        num_cores=NUM_CORES, num_subcores=NUM_SUBCORES,
    )

    @functools.partial(
        pl.kernel,
        out_type=jax.ShapeDtypeStruct((n_groups, GROUP, d), jnp.float32),
        mesh=mesh,
        scratch_types=[
            pltpu.VMEM((k, GROUP), jnp.int32),
            pltpu.VMEM((k, GROUP, d), jnp.float32),
            pltpu.SemaphoreType.DMA,
        ],
        compiler_params=pltpu.CompilerParams(use_tc_tiling_on_sc=False),
    )
    def gather_kernel(idx_hbm, table_hbm, out_hbm, idx_v, rows_v, sem):
        wid = lax.axis_index("s") * NUM_CORES + lax.axis_index("c")
        base = wid * groups_per_w

        def body(c, carry):
            g0 = base + c * k
            pltpu.sync_copy(idx_hbm.at[pl.ds(g0, k)], idx_v)
            copies = [
                pltpu.async_copy(table_hbm.at[idx_v.at[j]], rows_v.at[j], sem)
                for j in range(k)
            ]
            for cp in copies:
                cp.wait()
            pltpu.sync_copy(rows_v, out_hbm.at[pl.ds(g0, k)])
            return carry

        lax.fori_loop(0, n_chunks, body, 0)

    return gather_kernel(idx, table)


def kernel(x, table):
    batch, hist = x.shape
    _, d = table.shape
    total = batch * hist
    idx = x.reshape(total // GROUP, GROUP).astype(jnp.int32)
    out = _sc_gather(idx, table, k=8, d=d)
    return out.reshape(batch, hist, d)


# speedup vs baseline: 5.0538x; 1.0504x over previous
import functools

import jax
import jax.numpy as jnp
from jax import lax
from jax.experimental import pallas as pl
from jax.experimental.pallas import tpu as pltpu
from jax.experimental.pallas import tpu_sc as plsc

NUM_CORES = 2
NUM_SUBCORES = 16
NUM_WORKERS = NUM_CORES * NUM_SUBCORES
GROUP = 128
NBUF = 4


@functools.partial(jax.jit, static_argnames=("k", "d"))
def _sc_gather(idx, table, *, k, d):
    n_groups = idx.shape[0] // GROUP
    groups_per_w = n_groups // NUM_WORKERS
    n_chunks = groups_per_w // k
    n_rounds = n_chunks // NBUF
    assert n_chunks == n_rounds * NBUF and n_rounds >= 3

    mesh = plsc.VectorSubcoreMesh(
        core_axis_name="c", subcore_axis_name="s",
        num_cores=NUM_CORES, num_subcores=NUM_SUBCORES,
    )

    @functools.partial(
        pl.kernel,
        out_type=jax.ShapeDtypeStruct((n_groups, GROUP, d), jnp.float32),
        mesh=mesh,
        scratch_types=[
            [pltpu.VMEM((k * GROUP,), jnp.int32) for _ in range(NBUF)],
            [pltpu.VMEM((k, GROUP, d), jnp.float32) for _ in range(NBUF)],
            [pltpu.SemaphoreType.DMA for _ in range(NBUF)],
            [pltpu.SemaphoreType.DMA for _ in range(NBUF)],
            [pltpu.SemaphoreType.DMA for _ in range(NBUF)],
        ],
        compiler_params=pltpu.CompilerParams(use_tc_tiling_on_sc=False),
    )
    def gather_kernel(idx_hbm, table_hbm, out_hbm, ivs, rvs, isems, gsems, osems):
        wid = lax.axis_index("s") * NUM_CORES + lax.axis_index("c")
        base = wid * groups_per_w

        def fire_idx(c, q):
            pltpu.make_async_copy(
                idx_hbm.at[pl.ds((base + c * k) * GROUP, k * GROUP)],
                ivs[q], isems[q]).start()

        def wait_idx(q):
            pltpu.make_async_copy(
                idx_hbm.at[pl.ds(base * GROUP, k * GROUP)],
                ivs[q], isems[q]).wait()

        def fire_gathers(b):
            for j in range(k):
                pltpu.make_async_copy(
                    table_hbm.at[ivs[b].at[pl.ds(j * GROUP, GROUP)]],
                    rvs[b].at[j], gsems[b]).start()

        def wait_gathers(q):
            pltpu.make_async_copy(
                out_hbm.at[pl.ds(base, k)], rvs[q], gsems[q]).wait()

        def fire_write(c, q):
            pltpu.make_async_copy(
                rvs[q], out_hbm.at[pl.ds(base + c * k, k)], osems[q]).start()

        def wait_write(q):
            pltpu.make_async_copy(
                rvs[q], out_hbm.at[pl.ds(base, k)], osems[q]).wait()

        fire_idx(0, 0)
        for b in range(NBUF):
            if b >= 2:
                wait_gathers(b - 2)
                fire_write(b - 2, b - 2)
            wait_idx(b)
            fire_gathers(b)
            fire_idx(b + 1, (b + 1) % NBUF)

        def round_body(r, carry):
            v0 = r * NBUF
            for b in range(NBUF):
                v = v0 + b
                q = (b + 2) % NBUF
                wait_gathers(q)
                fire_write(v - 2, q)
                wait_write(b)
                wait_idx(b)
                fire_gathers(b)
                fire_idx(v + 1, (b + 1) % NBUF)
            return carry

        lax.fori_loop(1, n_rounds - 1, round_body, 0)

        v0 = (n_rounds - 1) * NBUF
        for b in range(NBUF):
            v = v0 + b
            q = (b + 2) % NBUF
            wait_gathers(q)
            fire_write(v - 2, q)
            wait_write(b)
            wait_idx(b)
            fire_gathers(b)
            if b + 1 < NBUF:
                fire_idx(v + 1, b + 1)

        n = n_chunks
        wait_gathers(2)
        fire_write(n - 2, 2)
        wait_gathers(3)
        fire_write(n - 1, 3)
        for q in range(NBUF):
            wait_write(q)

    return gather_kernel(idx, table)


def kernel(x, table):
    batch, hist = x.shape
    _, d = table.shape
    total = batch * hist
    idx = x.reshape(total).astype(jnp.int32)
    out = _sc_gather(idx, table, k=5, d=d)
    return out.reshape(batch, hist, d)
